# baseline (speedup 1.0000x reference)
import numpy as np
import jax
import jax.numpy as jnp
from jax.experimental import pallas as pl
from jax.experimental.pallas import tpu as pltpu

R, C = 32, 1000000
RG = 8
BLK = 32768
NBLK = (C + BLK - 1) // BLK

_K0 = np.uint32(928981903)
_K1 = np.uint32(3453687069)
_K2 = np.uint32(_K0 ^ _K1 ^ np.uint32(0x1BD11BDA))
_TINY = np.float32(np.finfo(np.float32).tiny)


def _threefry_bits(ctr):
    ks = (jnp.uint32(_K0), jnp.uint32(_K1), jnp.uint32(_K2))
    x0 = jnp.full_like(ctr, ks[0])
    x1 = ctr + ks[1]
    rots = ((13, 15, 26, 6), (17, 29, 16, 24))
    for i in range(5):
        for d in rots[i % 2]:
            x0 = x0 + x1
            x1 = ((x1 << jnp.uint32(d)) | (x1 >> jnp.uint32(32 - d))) ^ x0
        x0 = x0 + ks[(i + 1) % 3]
        x1 = x1 + ks[(i + 2) % 3] + jnp.uint32(i + 1)
    return x0 ^ x1


def _body(x_ref, oh_ref, pi_ref, s1_ref, m_ref, idx_ref):
    rg = pl.program_id(0)
    p = pl.program_id(1)
    j = pl.program_id(2)

    lane = jax.lax.broadcasted_iota(jnp.int32, (RG, BLK), 1)
    col = j * BLK + lane
    valid = col < C

    @pl.when(jnp.logical_and(p == 0, j == 0))
    def _init():
        s1_ref[...] = jnp.zeros((RG, 128), jnp.float32)
        m_ref[...] = jnp.full((RG, 128), -jnp.inf, jnp.float32)
        idx_ref[...] = jnp.zeros((RG, 128), jnp.int32)

    @pl.when(p == 0)
    def _phase0():
        x = x_ref[...]
        srow = jax.lax.broadcasted_iota(jnp.int32, (RG, BLK), 0)
        row = rg * RG + srow
        ctr = (row * C + col).astype(jnp.uint32)
        bits = _threefry_bits(ctr)
        fl = jax.lax.bitcast_convert_type(
            (bits >> jnp.uint32(9)) | jnp.uint32(0x3F800000), jnp.float32)
        u = jnp.maximum(fl - jnp.float32(1.0), _TINY)
        gum = -jnp.log(-jnp.log(u))
        g = jnp.where(valid, x + gum, -jnp.inf)

        bm = jnp.max(g, axis=1, keepdims=True)
        cand = jnp.where(g == bm, col, jnp.int32(2**30))
        bidx = jnp.min(cand, axis=1, keepdims=True)
        e = jnp.where(valid, jnp.exp(x), jnp.float32(0.0))
        bs = jnp.sum(e, axis=1, keepdims=True)

        bm_f = jnp.broadcast_to(bm, (RG, 128))
        bidx_f = jnp.broadcast_to(bidx, (RG, 128))
        upd = bm_f > m_ref[...]
        m_ref[...] = jnp.where(upd, bm_f, m_ref[...])
        idx_ref[...] = jnp.where(upd, bidx_f, idx_ref[...])
        s1_ref[...] = s1_ref[...] + jnp.broadcast_to(bs, (RG, 128))

    @pl.when(p == 1)
    def _phase1():
        x = x_ref[...]
        rec = jnp.float32(1.0) / s1_ref[:, 0:1]
        pi_ref[...] = jnp.exp(x) * rec
        oh_ref[...] = jnp.where(col == idx_ref[:, 0:1],
                                jnp.float32(1.0), jnp.float32(0.0))


def kernel(logits):
    grid = (R // RG, 2, NBLK)
    out = pl.pallas_call(
        _body,
        grid=grid,
        in_specs=[pl.BlockSpec((RG, BLK), lambda rg, p, j: (rg, j))],
        out_specs=[
            pl.BlockSpec((RG, BLK),
                         lambda rg, p, j: (rg, jnp.where(p == 0, 0, j))),
            pl.BlockSpec((RG, BLK),
                         lambda rg, p, j: (rg, jnp.where(p == 0, 0, j))),
        ],
        out_shape=[
            jax.ShapeDtypeStruct((R, C), jnp.float32),
            jax.ShapeDtypeStruct((R, C), jnp.float32),
        ],
        scratch_shapes=[
            pltpu.VMEM((RG, 128), jnp.float32),
            pltpu.VMEM((RG, 128), jnp.float32),
            pltpu.VMEM((RG, 128), jnp.int32),
        ],
        compiler_params=pltpu.CompilerParams(
            dimension_semantics=("arbitrary", "arbitrary", "arbitrary"),
        ),
    )(logits)
    one_hot, pi = out
    return (one_hot, pi)


# speedup vs baseline: 1.1548x; 1.1548x over previous
import numpy as np
import jax
import jax.numpy as jnp
from jax.experimental import pallas as pl
from jax.experimental.pallas import tpu as pltpu

R, C = 32, 1000000
RG = 8
BLK = 8192
NBLK = (C + BLK - 1) // BLK
TW = 512
NT = BLK // TW

_K0 = np.uint32(928981903)
_K1 = np.uint32(3453687069)
_K2 = np.uint32(_K0 ^ _K1 ^ np.uint32(0x1BD11BDA))
_TINY = np.float32(np.finfo(np.float32).tiny)


def _threefry_bits(ctr):
    ks = (jnp.uint32(_K0), jnp.uint32(_K1), jnp.uint32(_K2))
    x0 = jnp.full_like(ctr, ks[0])
    x1 = ctr + ks[1]
    rots = ((13, 15, 26, 6), (17, 29, 16, 24))
    for i in range(5):
        for d in rots[i % 2]:
            x0 = x0 + x1
            x1 = ((x1 << jnp.uint32(d)) | (x1 >> jnp.uint32(32 - d))) ^ x0
        x0 = x0 + ks[(i + 1) % 3]
        x1 = x1 + ks[(i + 2) % 3] + jnp.uint32(i + 1)
    return x0 ^ x1


def _body(x_ref, oh_ref, pi_ref, s1_ref, m_ref, idx_ref):
    rg = pl.program_id(0)
    p = pl.program_id(1)
    j = pl.program_id(2)

    lane = jax.lax.broadcasted_iota(jnp.int32, (RG, BLK), 1)
    col = j * BLK + lane
    valid = col < C

    @pl.when(jnp.logical_and(p == 0, j == 0))
    def _init():
        s1_ref[...] = jnp.zeros((RG, 128), jnp.float32)
        m_ref[...] = jnp.full((RG, 128), -jnp.inf, jnp.float32)
        idx_ref[...] = jnp.zeros((RG, 128), jnp.int32)

    @pl.when(p == 0)
    def _phase0():
        lane_t = jax.lax.broadcasted_iota(jnp.int32, (RG, TW), 1)
        srow = jax.lax.broadcasted_iota(jnp.int32, (RG, TW), 0)
        row_c = (rg * RG + srow) * C
        g_acc = jnp.full((RG, TW), -jnp.inf, jnp.float32)
        i_acc = jnp.zeros((RG, TW), jnp.int32)
        e_acc = jnp.zeros((RG, TW), jnp.float32)
        for t in range(NT):
            x = x_ref[:, t * TW:(t + 1) * TW]
            colt = j * BLK + t * TW + lane_t
            validt = colt < C
            ctr = (row_c + colt).astype(jnp.uint32)
            bits = _threefry_bits(ctr)
            fl = jax.lax.bitcast_convert_type(
                (bits >> jnp.uint32(9)) | jnp.uint32(0x3F800000), jnp.float32)
            u = jnp.maximum(fl - jnp.float32(1.0), _TINY)
            gum = -jnp.log(-jnp.log(u))
            g = jnp.where(validt, x + gum, -jnp.inf)
            upd = g > g_acc
            g_acc = jnp.maximum(g_acc, g)
            i_acc = jnp.where(upd, colt, i_acc)
            e_acc = e_acc + jnp.where(validt, jnp.exp(x), jnp.float32(0.0))

        bm = jnp.max(g_acc, axis=1, keepdims=True)
        cand = jnp.where(g_acc == bm, i_acc, jnp.int32(2**30))
        bidx = jnp.min(cand, axis=1, keepdims=True)
        bs = jnp.sum(e_acc, axis=1, keepdims=True)

        bm_f = jnp.broadcast_to(bm, (RG, 128))
        bidx_f = jnp.broadcast_to(bidx, (RG, 128))
        upd = bm_f > m_ref[...]
        m_ref[...] = jnp.where(upd, bm_f, m_ref[...])
        idx_ref[...] = jnp.where(upd, bidx_f, idx_ref[...])
        s1_ref[...] = s1_ref[...] + jnp.broadcast_to(bs, (RG, 128))

    @pl.when(p == 1)
    def _phase1():
        x = x_ref[...]
        rec = jnp.float32(1.0) / s1_ref[:, 0:1]
        pi_ref[...] = jnp.exp(x) * rec
        oh_ref[...] = jnp.where(col == idx_ref[:, 0:1],
                                jnp.float32(1.0), jnp.float32(0.0))


def kernel(logits):
    grid = (R // RG, 2, NBLK)
    out = pl.pallas_call(
        _body,
        grid=grid,
        in_specs=[pl.BlockSpec((RG, BLK), lambda rg, p, j: (rg, j))],
        out_specs=[
            pl.BlockSpec((RG, BLK),
                         lambda rg, p, j: (rg, jnp.where(p == 0, 0, j))),
            pl.BlockSpec((RG, BLK),
                         lambda rg, p, j: (rg, jnp.where(p == 0, 0, j))),
        ],
        out_shape=[
            jax.ShapeDtypeStruct((R, C), jnp.float32),
            jax.ShapeDtypeStruct((R, C), jnp.float32),
        ],
        scratch_shapes=[
            pltpu.VMEM((RG, 128), jnp.float32),
            pltpu.VMEM((RG, 128), jnp.float32),
            pltpu.VMEM((RG, 128), jnp.int32),
        ],
        compiler_params=pltpu.CompilerParams(
            dimension_semantics=("arbitrary", "arbitrary", "arbitrary"),
        ),
    )(logits)
    one_hot, pi = out
    return (one_hot, pi)


# speedup vs baseline: 1.4026x; 1.2145x over previous
import numpy as np
import jax
import jax.numpy as jnp
from jax.experimental import pallas as pl
from jax.experimental.pallas import tpu as pltpu

R, C = 32, 1000000
RG = 8
BLK = 16384
NBLK = (C + BLK - 1) // BLK
TW = 512
NT = BLK // TW

_K0 = np.uint32(928981903)
_K1 = np.uint32(3453687069)
_K2 = np.uint32(_K0 ^ _K1 ^ np.uint32(0x1BD11BDA))
_TINY = np.float32(np.finfo(np.float32).tiny)


def _threefry_bits(ctr):
    ks = (jnp.uint32(_K0), jnp.uint32(_K1), jnp.uint32(_K2))
    x0 = jnp.full_like(ctr, ks[0])
    x1 = ctr + ks[1]
    rots = ((13, 15, 26, 6), (17, 29, 16, 24))
    for i in range(5):
        for d in rots[i % 2]:
            x0 = x0 + x1
            x1 = ((x1 << jnp.uint32(d)) | (x1 >> jnp.uint32(32 - d))) ^ x0
        x0 = x0 + ks[(i + 1) % 3]
        x1 = x1 + ks[(i + 2) % 3] + jnp.uint32(i + 1)
    return x0 ^ x1


def _body(x_ref, oh_ref, pi_ref, s1_ref, m_ref, idx_ref):
    rg = pl.program_id(0)
    p = pl.program_id(1)
    j = pl.program_id(2)

    lane = jax.lax.broadcasted_iota(jnp.int32, (RG, BLK), 1)
    col = j * BLK + lane
    valid = col < C

    @pl.when(jnp.logical_and(p == 0, j == 0))
    def _init():
        s1_ref[...] = jnp.zeros((RG, 128), jnp.float32)
        m_ref[...] = jnp.full((RG, 128), -jnp.inf, jnp.float32)
        idx_ref[...] = jnp.zeros((RG, 128), jnp.int32)

    @pl.when(p == 0)
    def _phase0():
        lane_t = jax.lax.broadcasted_iota(jnp.int32, (RG, TW), 1)
        srow = jax.lax.broadcasted_iota(jnp.int32, (RG, TW), 0)
        row_c = (rg * RG + srow) * C
        g_acc = jnp.full((RG, TW), -jnp.inf, jnp.float32)
        i_acc = jnp.zeros((RG, TW), jnp.int32)
        e_acc = jnp.zeros((RG, TW), jnp.float32)
        for t in range(NT):
            colt = j * BLK + t * TW + lane_t
            x = jnp.where(colt < C, x_ref[:, t * TW:(t + 1) * TW],
                          -jnp.inf)
            ctr = (row_c + colt).astype(jnp.uint32)
            bits = _threefry_bits(ctr)
            fl = jax.lax.bitcast_convert_type(
                (bits >> jnp.uint32(9)) | jnp.uint32(0x3F800000), jnp.float32)
            u = jnp.maximum(fl - jnp.float32(1.0), _TINY)
            gum = -jnp.log(-jnp.log(u))
            g = x + gum
            upd = g > g_acc
            g_acc = jnp.maximum(g_acc, g)
            i_acc = jnp.where(upd, colt, i_acc)
            e_acc = e_acc + jnp.exp(x)

        bm = jnp.max(g_acc, axis=1, keepdims=True)
        cand = jnp.where(g_acc == bm, i_acc, jnp.int32(2**30))
        bidx = jnp.min(cand, axis=1, keepdims=True)
        bs = jnp.sum(e_acc, axis=1, keepdims=True)

        bm_f = jnp.broadcast_to(bm, (RG, 128))
        bidx_f = jnp.broadcast_to(bidx, (RG, 128))
        upd = bm_f > m_ref[...]
        m_ref[...] = jnp.where(upd, bm_f, m_ref[...])
        idx_ref[...] = jnp.where(upd, bidx_f, idx_ref[...])
        s1_ref[...] = s1_ref[...] + jnp.broadcast_to(bs, (RG, 128))

    @pl.when(p == 1)
    def _phase1():
        x = x_ref[...]
        rec = jnp.float32(1.0) / s1_ref[:, 0:1]
        pi_ref[...] = jnp.exp(x) * rec
        oh_ref[...] = jnp.where(col == idx_ref[:, 0:1],
                                jnp.float32(1.0), jnp.float32(0.0))


def kernel(logits):
    grid = (R // RG, 2, NBLK)
    out = pl.pallas_call(
        _body,
        grid=grid,
        in_specs=[pl.BlockSpec((RG, BLK), lambda rg, p, j: (rg, j))],
        out_specs=[
            pl.BlockSpec((RG, BLK),
                         lambda rg, p, j: (rg, jnp.where(p == 0, 0, j))),
            pl.BlockSpec((RG, BLK),
                         lambda rg, p, j: (rg, jnp.where(p == 0, 0, j))),
        ],
        out_shape=[
            jax.ShapeDtypeStruct((R, C), jnp.float32),
            jax.ShapeDtypeStruct((R, C), jnp.float32),
        ],
        scratch_shapes=[
            pltpu.VMEM((RG, 128), jnp.float32),
            pltpu.VMEM((RG, 128), jnp.float32),
            pltpu.VMEM((RG, 128), jnp.int32),
        ],
        compiler_params=pltpu.CompilerParams(
            dimension_semantics=("arbitrary", "arbitrary", "arbitrary"),
        ),
    )(logits)
    one_hot, pi = out
    return (one_hot, pi)


# speedup vs baseline: 1.5483x; 1.1039x over previous
import numpy as np
import jax
import jax.numpy as jnp
from jax.experimental import pallas as pl
from jax.experimental.pallas import tpu as pltpu

R, C = 32, 1000000
RG = 8
BLK = 32768
NBLK = (C + BLK - 1) // BLK
TW = 512
NT = BLK // TW

_K0 = np.uint32(928981903)
_K1 = np.uint32(3453687069)
_K2 = np.uint32(_K0 ^ _K1 ^ np.uint32(0x1BD11BDA))
_TINY = np.float32(np.finfo(np.float32).tiny)


def _threefry_bits(ctr):
    ks = (jnp.uint32(_K0), jnp.uint32(_K1), jnp.uint32(_K2))
    x0 = jnp.full_like(ctr, ks[0])
    x1 = ctr + ks[1]
    rots = ((13, 15, 26, 6), (17, 29, 16, 24))
    for i in range(5):
        for d in rots[i % 2]:
            x0 = x0 + x1
            x1 = ((x1 << jnp.uint32(d)) | (x1 >> jnp.uint32(32 - d))) ^ x0
        x0 = x0 + ks[(i + 1) % 3]
        x1 = x1 + ks[(i + 2) % 3] + jnp.uint32(i + 1)
    return x0 ^ x1


def _body(x_ref, oh_ref, pi_ref, s1_ref, m_ref, idx_ref):
    rg = pl.program_id(0)
    p = pl.program_id(1)
    j = pl.program_id(2)

    lane = jax.lax.broadcasted_iota(jnp.int32, (RG, BLK), 1)
    col = j * BLK + lane
    valid = col < C

    @pl.when(jnp.logical_and(p == 0, j == 0))
    def _init():
        s1_ref[...] = jnp.zeros((RG, 128), jnp.float32)
        m_ref[...] = jnp.full((RG, 128), -jnp.inf, jnp.float32)
        idx_ref[...] = jnp.zeros((RG, 128), jnp.int32)

    @pl.when(p == 0)
    def _phase0():
        lane_t = jax.lax.broadcasted_iota(jnp.int32, (RG, TW), 1)
        srow = jax.lax.broadcasted_iota(jnp.int32, (RG, TW), 0)
        row_c = (rg * RG + srow) * C
        g_acc = jnp.full((RG, TW), -jnp.inf, jnp.float32)
        i_acc = jnp.zeros((RG, TW), jnp.int32)
        e_acc = jnp.zeros((RG, TW), jnp.float32)
        for t in range(NT):
            colt = j * BLK + t * TW + lane_t
            x = jnp.where(colt < C, x_ref[:, t * TW:(t + 1) * TW],
                          -jnp.inf)
            ctr = (row_c + colt).astype(jnp.uint32)
            bits = _threefry_bits(ctr)
            fl = jax.lax.bitcast_convert_type(
                (bits >> jnp.uint32(9)) | jnp.uint32(0x3F800000), jnp.float32)
            u = jnp.maximum(fl - jnp.float32(1.0), _TINY)
            gum = -jnp.log(-jnp.log(u))
            g = x + gum
            upd = g > g_acc
            g_acc = jnp.maximum(g_acc, g)
            i_acc = jnp.where(upd, colt, i_acc)
            e_acc = e_acc + jnp.exp(x)

        bm = jnp.max(g_acc, axis=1, keepdims=True)
        cand = jnp.where(g_acc == bm, i_acc, jnp.int32(2**30))
        bidx = jnp.min(cand, axis=1, keepdims=True)
        bs = jnp.sum(e_acc, axis=1, keepdims=True)

        bm_f = jnp.broadcast_to(bm, (RG, 128))
        bidx_f = jnp.broadcast_to(bidx, (RG, 128))
        upd = bm_f > m_ref[...]
        m_ref[...] = jnp.where(upd, bm_f, m_ref[...])
        idx_ref[...] = jnp.where(upd, bidx_f, idx_ref[...])
        s1_ref[...] = s1_ref[...] + jnp.broadcast_to(bs, (RG, 128))

    @pl.when(p == 1)
    def _phase1():
        x = x_ref[...]
        rec = jnp.float32(1.0) / s1_ref[:, 0:1]
        pi_ref[...] = jnp.exp(x) * rec
        oh_ref[...] = jnp.where(col == idx_ref[:, 0:1],
                                jnp.float32(1.0), jnp.float32(0.0))


def kernel(logits):
    grid = (R // RG, 2, NBLK)
    out = pl.pallas_call(
        _body,
        grid=grid,
        in_specs=[pl.BlockSpec((RG, BLK), lambda rg, p, j: (rg, j))],
        out_specs=[
            pl.BlockSpec((RG, BLK),
                         lambda rg, p, j: (rg, jnp.where(p == 0, 0, j))),
            pl.BlockSpec((RG, BLK),
                         lambda rg, p, j: (rg, jnp.where(p == 0, 0, j))),
        ],
        out_shape=[
            jax.ShapeDtypeStruct((R, C), jnp.float32),
            jax.ShapeDtypeStruct((R, C), jnp.float32),
        ],
        scratch_shapes=[
            pltpu.VMEM((RG, 128), jnp.float32),
            pltpu.VMEM((RG, 128), jnp.float32),
            pltpu.VMEM((RG, 128), jnp.int32),
        ],
        compiler_params=pltpu.CompilerParams(
            dimension_semantics=("arbitrary", "arbitrary", "arbitrary"),
        ),
    )(logits)
    one_hot, pi = out
    return (one_hot, pi)


# speedup vs baseline: 1.7788x; 1.1489x over previous
import numpy as np
import jax
import jax.numpy as jnp
from jax.experimental import pallas as pl
from jax.experimental.pallas import tpu as pltpu

R, C = 32, 1000000
RG = 8
RGN = R // RG
BLK = 32768
NBLK = (C + BLK - 1) // BLK
TW = 512
NT = BLK // TW

_K0 = np.uint32(928981903)
_K1 = np.uint32(3453687069)
_K2 = np.uint32(_K0 ^ _K1 ^ np.uint32(0x1BD11BDA))
_TINY = np.float32(np.finfo(np.float32).tiny)


def _threefry_bits(ctr):
    ks = (jnp.uint32(_K0), jnp.uint32(_K1), jnp.uint32(_K2))
    x0 = jnp.full_like(ctr, ks[0])
    x1 = ctr + ks[1]
    rots = ((13, 15, 26, 6), (17, 29, 16, 24))
    for i in range(5):
        for d in rots[i % 2]:
            x0 = x0 + x1
            x1 = ((x1 << jnp.uint32(d)) | (x1 >> jnp.uint32(32 - d))) ^ x0
        x0 = x0 + ks[(i + 1) % 3]
        x1 = x1 + ks[(i + 2) % 3] + jnp.uint32(i + 1)
    return x0 ^ x1


def _body(xc_ref, xw_ref, oh_ref, pi_ref,
          s1_ref, m_ref, idx_ref, s1p_ref, idxp_ref):
    rg = pl.program_id(0)
    j = pl.program_id(1)

    @pl.when(j == 0)
    def _roll():
        s1p_ref[...] = s1_ref[...]
        idxp_ref[...] = idx_ref[...]
        s1_ref[...] = jnp.zeros((RG, 128), jnp.float32)
        m_ref[...] = jnp.full((RG, 128), -jnp.inf, jnp.float32)
        idx_ref[...] = jnp.zeros((RG, 128), jnp.int32)

    @pl.when(rg < RGN)
    def _reduce():
        lane_t = jax.lax.broadcasted_iota(jnp.int32, (RG, TW), 1)
        srow = jax.lax.broadcasted_iota(jnp.int32, (RG, TW), 0)
        row_c = (rg * RG + srow) * C
        g_acc = jnp.full((RG, TW), -jnp.inf, jnp.float32)
        i_acc = jnp.zeros((RG, TW), jnp.int32)
        e_acc = jnp.zeros((RG, TW), jnp.float32)
        for t in range(NT):
            colt = j * BLK + t * TW + lane_t
            x = jnp.where(colt < C, xc_ref[:, t * TW:(t + 1) * TW],
                          -jnp.inf)
            ctr = (row_c + colt).astype(jnp.uint32)
            bits = _threefry_bits(ctr)
            fl = jax.lax.bitcast_convert_type(
                (bits >> jnp.uint32(9)) | jnp.uint32(0x3F800000), jnp.float32)
            u = jnp.maximum(fl - jnp.float32(1.0), _TINY)
            gum = -jnp.log(-jnp.log(u))
            g = x + gum
            upd = g > g_acc
            g_acc = jnp.maximum(g_acc, g)
            i_acc = jnp.where(upd, colt, i_acc)
            e_acc = e_acc + jnp.exp(x)

        bm = jnp.max(g_acc, axis=1, keepdims=True)
        cand = jnp.where(g_acc == bm, i_acc, jnp.int32(2**30))
        bidx = jnp.min(cand, axis=1, keepdims=True)
        bs = jnp.sum(e_acc, axis=1, keepdims=True)

        bm_f = jnp.broadcast_to(bm, (RG, 128))
        bidx_f = jnp.broadcast_to(bidx, (RG, 128))
        upd = bm_f > m_ref[...]
        m_ref[...] = jnp.where(upd, bm_f, m_ref[...])
        idx_ref[...] = jnp.where(upd, bidx_f, idx_ref[...])
        s1_ref[...] = s1_ref[...] + jnp.broadcast_to(bs, (RG, 128))

    @pl.when(rg > 0)
    def _write():
        xw = xw_ref[...]
        col = j * BLK + jax.lax.broadcasted_iota(jnp.int32, (RG, BLK), 1)
        rec = jnp.float32(1.0) / s1p_ref[:, 0:1]
        pi_ref[...] = jnp.exp(xw) * rec
        oh_ref[...] = jnp.where(col == idxp_ref[:, 0:1],
                                jnp.float32(1.0), jnp.float32(0.0))


def kernel(logits):
    grid = (RGN + 1, NBLK)
    out = pl.pallas_call(
        _body,
        grid=grid,
        in_specs=[
            pl.BlockSpec((RG, BLK),
                         lambda rg, j: (jnp.minimum(rg, RGN - 1),
                                        jnp.where(rg < RGN, j, NBLK - 1))),
            pl.BlockSpec((RG, BLK),
                         lambda rg, j: (jnp.maximum(rg - 1, 0),
                                        jnp.where(rg > 0, j, 0))),
        ],
        out_specs=[
            pl.BlockSpec((RG, BLK),
                         lambda rg, j: (jnp.maximum(rg - 1, 0),
                                        jnp.where(rg > 0, j, 0))),
            pl.BlockSpec((RG, BLK),
                         lambda rg, j: (jnp.maximum(rg - 1, 0),
                                        jnp.where(rg > 0, j, 0))),
        ],
        out_shape=[
            jax.ShapeDtypeStruct((R, C), jnp.float32),
            jax.ShapeDtypeStruct((R, C), jnp.float32),
        ],
        scratch_shapes=[
            pltpu.VMEM((RG, 128), jnp.float32),
            pltpu.VMEM((RG, 128), jnp.float32),
            pltpu.VMEM((RG, 128), jnp.int32),
            pltpu.VMEM((RG, 128), jnp.float32),
            pltpu.VMEM((RG, 128), jnp.int32),
        ],
        compiler_params=pltpu.CompilerParams(
            dimension_semantics=("arbitrary", "arbitrary"),
        ),
    )(logits, logits)
    one_hot, pi = out
    return (one_hot, pi)


# speedup vs baseline: 1.8542x; 1.0424x over previous
import numpy as np
import jax
import jax.numpy as jnp
from jax.experimental import pallas as pl
from jax.experimental.pallas import tpu as pltpu

R, C = 32, 1000000
RG = 8
RGN = R // RG
BLK = 50176
NBLK = (C + BLK - 1) // BLK
TW = 512
NT = BLK // TW

_K0 = np.uint32(928981903)
_K1 = np.uint32(3453687069)
_K2 = np.uint32(_K0 ^ _K1 ^ np.uint32(0x1BD11BDA))
_TINY = np.float32(np.finfo(np.float32).tiny)


def _threefry_bits(x1):
    ks = (jnp.uint32(_K0), jnp.uint32(_K1), jnp.uint32(_K2))
    x0 = jnp.full_like(x1, ks[0])
    rots = ((13, 15, 26, 6), (17, 29, 16, 24))
    for i in range(5):
        for d in rots[i % 2]:
            x0 = x0 + x1
            x1 = ((x1 << jnp.uint32(d)) | (x1 >> jnp.uint32(32 - d))) ^ x0
        x0 = x0 + ks[(i + 1) % 3]
        x1 = x1 + ks[(i + 2) % 3] + jnp.uint32(i + 1)
    return x0 ^ x1


def _body(xc_ref, xw_ref, oh_ref, pi_ref,
          s1_ref, m_ref, idx_ref, s1p_ref, idxp_ref):
    rg = pl.program_id(0)
    j = pl.program_id(1)

    @pl.when(j == 0)
    def _roll():
        s1p_ref[...] = s1_ref[...]
        idxp_ref[...] = idx_ref[...]
        s1_ref[...] = jnp.zeros((RG, 128), jnp.float32)
        m_ref[...] = jnp.full((RG, 128), -jnp.inf, jnp.float32)
        idx_ref[...] = jnp.zeros((RG, 128), jnp.int32)

    @pl.when(rg < RGN)
    def _reduce():
        lane_t = jax.lax.broadcasted_iota(jnp.int32, (RG, TW), 1)
        srow = jax.lax.broadcasted_iota(jnp.int32, (RG, TW), 0)
        row_c = (rg * RG + srow) * C
        row_ck = row_c.astype(jnp.uint32) + jnp.uint32(_K1)
        g_acc = jnp.full((RG, TW), -jnp.inf, jnp.float32)
        i_acc = jnp.zeros((RG, TW), jnp.int32)
        e_acc = jnp.zeros((RG, TW), jnp.float32)
        for t in range(NT):
            colt = j * BLK + t * TW + lane_t
            x = jnp.where(colt < C, xc_ref[:, t * TW:(t + 1) * TW],
                          -jnp.inf)
            bits = _threefry_bits(row_ck + colt.astype(jnp.uint32))
            fl = jax.lax.bitcast_convert_type(
                (bits >> jnp.uint32(9)) | jnp.uint32(0x3F800000), jnp.float32)
            u = jnp.maximum(fl - jnp.float32(1.0), _TINY)
            gum = -jnp.log(-jnp.log(u))
            g = x + gum
            upd = g > g_acc
            g_acc = jnp.maximum(g_acc, g)
            i_acc = jnp.where(upd, colt, i_acc)
            e_acc = e_acc + jnp.exp(x)

        bm = jnp.max(g_acc, axis=1, keepdims=True)
        cand = jnp.where(g_acc == bm, i_acc, jnp.int32(2**30))
        bidx = jnp.min(cand, axis=1, keepdims=True)
        bs = jnp.sum(e_acc, axis=1, keepdims=True)

        bm_f = jnp.broadcast_to(bm, (RG, 128))
        bidx_f = jnp.broadcast_to(bidx, (RG, 128))
        upd = bm_f > m_ref[...]
        m_ref[...] = jnp.where(upd, bm_f, m_ref[...])
        idx_ref[...] = jnp.where(upd, bidx_f, idx_ref[...])
        s1_ref[...] = s1_ref[...] + jnp.broadcast_to(bs, (RG, 128))

    @pl.when(rg > 0)
    def _write():
        xw = xw_ref[...]
        col = j * BLK + jax.lax.broadcasted_iota(jnp.int32, (RG, BLK), 1)
        rec = jnp.float32(1.0) / s1p_ref[:, 0:1]
        pi_ref[...] = jnp.exp(xw) * rec
        oh_ref[...] = jnp.where(col == idxp_ref[:, 0:1],
                                jnp.float32(1.0), jnp.float32(0.0))


def kernel(logits):
    grid = (RGN + 1, NBLK)
    out = pl.pallas_call(
        _body,
        grid=grid,
        in_specs=[
            pl.BlockSpec((RG, BLK),
                         lambda rg, j: (jnp.minimum(rg, RGN - 1),
                                        jnp.where(rg < RGN, j, NBLK - 1))),
            pl.BlockSpec((RG, BLK),
                         lambda rg, j: (jnp.maximum(rg - 1, 0),
                                        jnp.where(rg > 0, j, 0))),
        ],
        out_specs=[
            pl.BlockSpec((RG, BLK),
                         lambda rg, j: (jnp.maximum(rg - 1, 0),
                                        jnp.where(rg > 0, j, 0))),
            pl.BlockSpec((RG, BLK),
                         lambda rg, j: (jnp.maximum(rg - 1, 0),
                                        jnp.where(rg > 0, j, 0))),
        ],
        out_shape=[
            jax.ShapeDtypeStruct((R, C), jnp.float32),
            jax.ShapeDtypeStruct((R, C), jnp.float32),
        ],
        scratch_shapes=[
            pltpu.VMEM((RG, 128), jnp.float32),
            pltpu.VMEM((RG, 128), jnp.float32),
            pltpu.VMEM((RG, 128), jnp.int32),
            pltpu.VMEM((RG, 128), jnp.float32),
            pltpu.VMEM((RG, 128), jnp.int32),
        ],
        compiler_params=pltpu.CompilerParams(
            dimension_semantics=("arbitrary", "arbitrary"),
        ),
    )(logits, logits)
    one_hot, pi = out
    return (one_hot, pi)


# speedup vs baseline: 1.8587x; 1.0025x over previous
import numpy as np
import jax
import jax.numpy as jnp
from jax.experimental import pallas as pl
from jax.experimental.pallas import tpu as pltpu

R, C = 32, 1000000
RG = 8
RGN = R // RG
BLK = 50176
NBLK = (C + BLK - 1) // BLK
TW = 512
NT = BLK // TW

_K0 = np.uint32(928981903)
_K1 = np.uint32(3453687069)
_K2 = np.uint32(_K0 ^ _K1 ^ np.uint32(0x1BD11BDA))
_TINY = np.float32(np.finfo(np.float32).tiny)


def _threefry_bits(x1):
    ks = (jnp.uint32(_K0), jnp.uint32(_K1), jnp.uint32(_K2))
    x0 = jnp.full_like(x1, ks[0])
    rots = ((13, 15, 26, 6), (17, 29, 16, 24))
    for i in range(5):
        for d in rots[i % 2]:
            x0 = x0 + x1
            x1 = ((x1 << jnp.uint32(d)) | (x1 >> jnp.uint32(32 - d))) ^ x0
        x0 = x0 + ks[(i + 1) % 3]
        x1 = x1 + ks[(i + 2) % 3] + jnp.uint32(i + 1)
    return x0 ^ x1


def _body(xc_ref, xw_ref, oh_ref, pi_ref,
          s1_ref, m_ref, idx_ref, s1p_ref, idxp_ref):
    rg = pl.program_id(0)
    j = pl.program_id(1)

    @pl.when(j == 0)
    def _roll():
        s1p_ref[...] = s1_ref[...]
        idxp_ref[...] = idx_ref[...]
        s1_ref[...] = jnp.zeros((RG, 128), jnp.float32)
        m_ref[...] = jnp.full((RG, 128), -jnp.inf, jnp.float32)
        idx_ref[...] = jnp.zeros((RG, 128), jnp.int32)

    @pl.when(rg < RGN)
    def _reduce():
        lane_t = jax.lax.broadcasted_iota(jnp.int32, (RG, TW), 1)
        srow = jax.lax.broadcasted_iota(jnp.int32, (RG, TW), 0)
        row_c = (rg * RG + srow) * C
        row_ck = row_c.astype(jnp.uint32) + jnp.uint32(_K1)
        g_acc = jnp.full((RG, TW), -jnp.inf, jnp.float32)
        i_acc = jnp.zeros((RG, TW), jnp.int32)
        e_acc = jnp.zeros((RG, TW), jnp.float32)
        for t in range(NT):
            colt = j * BLK + t * TW + lane_t
            x = xc_ref[:, t * TW:(t + 1) * TW]
            if (NBLK - 1) * BLK + (t + 1) * TW > C:
                x = jnp.where(colt < C, x, -jnp.inf)
            bits = _threefry_bits(row_ck + colt.astype(jnp.uint32))
            fl = jax.lax.bitcast_convert_type(
                (bits >> jnp.uint32(9)) | jnp.uint32(0x3F800000), jnp.float32)
            u = jnp.maximum(fl - jnp.float32(1.0), _TINY)
            gum = -jnp.log(-jnp.log(u))
            g = x + gum
            upd = g > g_acc
            g_acc = jnp.maximum(g_acc, g)
            i_acc = jnp.where(upd, colt, i_acc)
            e_acc = e_acc + jnp.exp(x)

        bm = jnp.max(g_acc, axis=1, keepdims=True)
        cand = jnp.where(g_acc == bm, i_acc, jnp.int32(2**30))
        bidx = jnp.min(cand, axis=1, keepdims=True)
        bs = jnp.sum(e_acc, axis=1, keepdims=True)

        bm_f = jnp.broadcast_to(bm, (RG, 128))
        bidx_f = jnp.broadcast_to(bidx, (RG, 128))
        upd = bm_f > m_ref[...]
        m_ref[...] = jnp.where(upd, bm_f, m_ref[...])
        idx_ref[...] = jnp.where(upd, bidx_f, idx_ref[...])
        s1_ref[...] = s1_ref[...] + jnp.broadcast_to(bs, (RG, 128))

    @pl.when(rg > 0)
    def _write():
        xw = xw_ref[...]
        lane = jax.lax.broadcasted_iota(jnp.int32, (RG, BLK), 1)
        rec = jnp.float32(1.0) / s1p_ref[:, 0:1]
        pi_ref[...] = jnp.exp(xw) * rec
        oh_ref[...] = jnp.where(lane == idxp_ref[:, 0:1] - j * BLK,
                                jnp.float32(1.0), jnp.float32(0.0))


def kernel(logits):
    grid = (RGN + 1, NBLK)
    out = pl.pallas_call(
        _body,
        grid=grid,
        in_specs=[
            pl.BlockSpec((RG, BLK),
                         lambda rg, j: (jnp.minimum(rg, RGN - 1),
                                        jnp.where(rg < RGN, j, NBLK - 1))),
            pl.BlockSpec((RG, BLK),
                         lambda rg, j: (jnp.maximum(rg - 1, 0),
                                        jnp.where(rg > 0, j, 0))),
        ],
        out_specs=[
            pl.BlockSpec((RG, BLK),
                         lambda rg, j: (jnp.maximum(rg - 1, 0),
                                        jnp.where(rg > 0, j, 0))),
            pl.BlockSpec((RG, BLK),
                         lambda rg, j: (jnp.maximum(rg - 1, 0),
                                        jnp.where(rg > 0, j, 0))),
        ],
        out_shape=[
            jax.ShapeDtypeStruct((R, C), jnp.float32),
            jax.ShapeDtypeStruct((R, C), jnp.float32),
        ],
        scratch_shapes=[
            pltpu.VMEM((RG, 128), jnp.float32),
            pltpu.VMEM((RG, 128), jnp.float32),
            pltpu.VMEM((RG, 128), jnp.int32),
            pltpu.VMEM((RG, 128), jnp.float32),
            pltpu.VMEM((RG, 128), jnp.int32),
        ],
        compiler_params=pltpu.CompilerParams(
            dimension_semantics=("arbitrary", "arbitrary"),
        ),
    )(logits, logits)
    one_hot, pi = out
    return (one_hot, pi)
